# baseline (speedup 1.0000x reference)
import functools

import jax
import jax.numpy as jnp
from jax import lax
from jax.experimental import pallas as pl
from jax.experimental.pallas import tpu as pltpu
from jax.experimental.pallas import tpu_sc as plsc

D = 32
SEQ = 200
LANES = 16

_info = plsc.get_sparse_core_info()
_NC, _NS = _info.num_cores, _info.num_subcores
_NW = _NC * _NS

_BB = 512
_GS = 128
_NBUF = 3


@functools.partial(jax.jit, static_argnums=(0, 1))
def _embed(batch, seq, x_lin, table, pos):
    nblk = batch // _BB
    n_units = seq * nblk
    upw = n_units // _NW

    mesh = plsc.VectorSubcoreMesh(core_axis_name="c", subcore_axis_name="s")

    scratch = (
        [pltpu.VMEM((_BB,), jnp.int32) for _ in range(_NBUF)]
        + [pltpu.VMEM((_BB, D), jnp.float32) for _ in range(_NBUF)]
        + [pltpu.VMEM((D * _BB,), jnp.float32) for _ in range(_NBUF)]
        + [pltpu.VMEM((SEQ, D), jnp.float32)]
        + [pltpu.SemaphoreType.DMA for _ in range(3 * _NBUF)]
    )

    @functools.partial(
        pl.kernel,
        mesh=mesh,
        out_type=jax.ShapeDtypeStruct((seq, D, batch), jnp.float32),
        compiler_params=pltpu.CompilerParams(
            use_tc_tiling_on_sc=False, needs_layout_passes=False),
        scratch_types=scratch,
    )
    def k(x_hbm, tab_hbm, pos_hbm, out_hbm, *sc):
        idx = sc[:_NBUF]
        rows = sc[_NBUF:2 * _NBUF]
        blk = sc[2 * _NBUF:3 * _NBUF]
        pos_v = sc[3 * _NBUF]
        sems = sc[3 * _NBUF + 1:]
        isem = sems[:_NBUF]
        gsem = sems[_NBUF:2 * _NBUF]
        osem = sems[2 * _NBUF:]

        wid = lax.axis_index("s") * _NC + lax.axis_index("c")
        pltpu.sync_copy(pos_hbm, pos_v)

        iota = lax.iota(jnp.int32, LANES)
        sidx0 = iota * _BB
        sidx1 = sidx0 + LANES * _BB

        idx_d, gat_d, out_d = {}, {}, {}

        def unit_lb(u):
            g = u * _NW + wid
            l = g // nblk
            b0 = (g % nblk) * _BB
            return l, b0

        def fire_idx(u):
            b = u % _NBUF
            l, b0 = unit_lb(u)
            t0 = pl.multiple_of(l * batch + b0, _BB)
            idx_d[u] = pltpu.async_copy(
                x_hbm.at[pl.ds(t0, _BB)], idx[b], isem[b])

        def fire_gathers(u):
            b = u % _NBUF
            idx_d.pop(u).wait()
            gat_d[u] = [
                pltpu.async_copy(
                    tab_hbm.at[idx[b].at[pl.ds(j * _GS, _GS)]],
                    rows[b].at[pl.ds(j * _GS, _GS)],
                    gsem[b],
                )
                for j in range(_BB // _GS)
            ]

        def transform(u):
            b = u % _NBUF
            rb, ob = rows[b], blk[b]
            l, _ = unit_lb(u)
            pv0 = pos_v[l, pl.ds(0, LANES)]
            pv1 = pos_v[l, pl.ds(LANES, LANES)]

            def body(j, c):
                v0 = rb[j, pl.ds(0, LANES)] + pv0
                v1 = rb[j, pl.ds(LANES, LANES)] + pv1
                plsc.store_scatter(ob, [sidx0 + j], v0)
                plsc.store_scatter(ob, [sidx1 + j], v1)
                return c

            lax.fori_loop(0, _BB, body, 0)

        def fire_out(u):
            b = u % _NBUF
            l, b0 = unit_lb(u)

            def fires(d, c):
                pltpu.async_copy(
                    blk[b].at[pl.ds(d * _BB, _BB)],
                    out_hbm.at[l, d, pl.ds(b0, _BB)],
                    osem[b],
                )
                return c

            lax.fori_loop(0, D, fires, 0)

        def wait_out(u):
            b = u % _NBUF
            pltpu.make_async_copy(
                tab_hbm.at[pl.ds(0, _BB)], rows[b], osem[b]).wait()

        for u in range(min(2, upw)):
            fire_idx(u)
        if upw > 0:
            fire_gathers(0)

        for u in range(upw):
            if u + 2 < upw:
                fire_idx(u + 2)
            if u + 1 < upw:
                fire_gathers(u + 1)
            for d in gat_d.pop(u):
                d.wait()
            if u >= _NBUF:
                wait_out(u - _NBUF)
            transform(u)
            fire_out(u)

        for u in range(max(0, upw - _NBUF), upw):
            wait_out(u)

    return k(x_lin, table, pos)


def kernel(x, embedding_table, possitional_emb):
    b, l = x.shape
    x_lin = x.T.reshape(b * l)
    out_t = _embed(b, l, x_lin, embedding_table, possitional_emb)
    return out_t.transpose(2, 0, 1)


# speedup vs baseline: 1.0022x; 1.0022x over previous
import functools

import jax
import jax.numpy as jnp
from jax import lax
from jax.experimental import pallas as pl
from jax.experimental.pallas import tpu as pltpu
from jax.experimental.pallas import tpu_sc as plsc

D = 32
SEQ = 200
LANES = 16

_info = plsc.get_sparse_core_info()
_NC, _NS = _info.num_cores, _info.num_subcores
_NW = _NC * _NS

_BB = 512
_GS = 128
_NB = 2
_UNROLL = 8


@functools.partial(jax.jit, static_argnums=(0, 1))
def _embed(batch, seq, x_lin, table, pos):
    nblk = batch // _BB
    n_units = seq * nblk
    upw = n_units // _NW
    assert upw % _NB == 0 and upw >= 4

    mesh = plsc.VectorSubcoreMesh(core_axis_name="c", subcore_axis_name="s")

    scratch = (
        [pltpu.VMEM((_BB,), jnp.int32) for _ in range(_NB)]
        + [pltpu.VMEM((_BB, D), jnp.float32) for _ in range(_NB)]
        + [pltpu.VMEM((D * _BB,), jnp.float32) for _ in range(_NB)]
        + [pltpu.VMEM((SEQ, D), jnp.float32)]
        + [pltpu.SemaphoreType.DMA for _ in range(3 * _NB)]
    )

    @functools.partial(
        pl.kernel,
        mesh=mesh,
        out_type=jax.ShapeDtypeStruct((seq, D, batch), jnp.float32),
        compiler_params=pltpu.CompilerParams(
            use_tc_tiling_on_sc=False, needs_layout_passes=False),
        scratch_types=scratch,
    )
    def k(x_hbm, tab_hbm, pos_hbm, out_hbm, *sc):
        idx = sc[:_NB]
        rows = sc[_NB:2 * _NB]
        blk = sc[2 * _NB:3 * _NB]
        pos_v = sc[3 * _NB]
        sems = sc[3 * _NB + 1:]
        isem = sems[:_NB]
        gsem = sems[_NB:2 * _NB]
        osem = sems[2 * _NB:]

        wid = lax.axis_index("s") * _NC + lax.axis_index("c")
        pltpu.sync_copy(pos_hbm, pos_v)

        iota = lax.iota(jnp.int32, LANES)
        sidx0 = iota * _BB
        off = LANES * _BB

        def unit_lb(u):
            g = u * _NW + wid
            return g // nblk, (g % nblk) * _BB

        def fire_idx(u, b):
            l, b0 = unit_lb(u)
            t0 = pl.multiple_of(l * batch + b0, _BB)
            pltpu.async_copy(x_hbm.at[pl.ds(t0, _BB)], idx[b], isem[b])

        def wait_idx(b):
            pltpu.make_async_copy(
                x_hbm.at[pl.ds(0, _BB)], idx[b], isem[b]).wait()

        def fire_gathers(b):
            for j in range(_BB // _GS):
                pltpu.async_copy(
                    tab_hbm.at[idx[b].at[pl.ds(j * _GS, _GS)]],
                    rows[b].at[pl.ds(j * _GS, _GS)],
                    gsem[b],
                )

        def wait_gathers(b):
            pltpu.make_async_copy(
                tab_hbm.at[pl.ds(0, _BB)], rows[b], gsem[b]).wait()

        def transform(u, b):
            rb, ob = rows[b], blk[b]
            l, _ = unit_lb(u)
            pv0 = pos_v[l, pl.ds(0, LANES)]
            pv1 = pos_v[l, pl.ds(LANES, LANES)]

            def body(g, sidx):
                j0 = g * _UNROLL
                for t in range(_UNROLL):
                    j = j0 + t
                    v0 = rb[j, pl.ds(0, LANES)] + pv0
                    v1 = rb[j, pl.ds(LANES, LANES)] + pv1
                    plsc.store_scatter(ob, [sidx], v0)
                    plsc.store_scatter(ob, [sidx + off], v1)
                    sidx = sidx + 1
                return sidx

            lax.fori_loop(0, _BB // _UNROLL, body, sidx0)

        def fire_out(u, b):
            l, b0 = unit_lb(u)

            def fires(d, c):
                pltpu.async_copy(
                    blk[b].at[pl.ds(d * _BB, _BB)],
                    out_hbm.at[l, d, pl.ds(b0, _BB)],
                    osem[b],
                )
                return c

            lax.fori_loop(0, D, fires, 0)

        def wait_out(b):
            pltpu.make_async_copy(
                tab_hbm.at[pl.ds(0, _BB)], rows[b], osem[b]).wait()

        def step(u, b, prefetch_idx, first, last):
            wait_gathers(b)
            if prefetch_idx:
                fire_idx(u + 2, b)
            if not last:
                wait_idx(1 - b)
                fire_gathers(1 - b)
            if not first:
                wait_out(b)
            transform(u, b)
            fire_out(u, b)

        fire_idx(0, 0)
        fire_idx(1, 1)
        wait_idx(0)
        fire_gathers(0)

        def main(it, c):
            u0 = it * _NB
            step(u0, 0, True, False, False)
            step(u0 + 1, 1, True, False, False)
            return c

        step(0, 0, True, True, False)
        step(1, 1, True, True, False)
        lax.fori_loop(1, (upw - 2) // _NB, main, 0)
        step(upw - 2, 0, False, False, False)
        step(upw - 1, 1, False, False, True)
        wait_out(0)
        wait_out(1)

    return k(x_lin, table, pos)


def kernel(x, embedding_table, possitional_emb):
    b, l = x.shape
    x_lin = x.T.reshape(b * l)
    out_t = _embed(b, l, x_lin, embedding_table, possitional_emb)
    return out_t.transpose(2, 0, 1)


# speedup vs baseline: 1.4996x; 1.4964x over previous
import functools

import jax
import jax.numpy as jnp
from jax import lax
from jax.experimental import pallas as pl
from jax.experimental.pallas import tpu as pltpu
from jax.experimental.pallas import tpu_sc as plsc

D = 32
SEQ = 200
LANES = 16

_info = plsc.get_sparse_core_info()
_NC, _NS = _info.num_cores, _info.num_subcores
_NW = _NC * _NS

_BB = 512
_GS = 128
_NB = 2
_PITCH = _BB + 8
_UNROLL = 8


@functools.partial(jax.jit, static_argnums=(0, 1))
def _embed(batch, seq, x_lin, table, pos):
    nblk = batch // _BB
    n_units = seq * nblk
    upw = n_units // _NW
    assert upw % _NB == 0 and upw >= 4

    mesh = plsc.VectorSubcoreMesh(core_axis_name="c", subcore_axis_name="s")

    scratch = (
        [pltpu.VMEM((_BB,), jnp.int32) for _ in range(_NB)]
        + [pltpu.VMEM((_BB, D), jnp.float32) for _ in range(_NB)]
        + [pltpu.VMEM((D * _PITCH,), jnp.float32) for _ in range(_NB)]
        + [pltpu.VMEM((SEQ, D), jnp.float32)]
        + [pltpu.SemaphoreType.DMA for _ in range(3 * _NB)]
    )

    @functools.partial(
        pl.kernel,
        mesh=mesh,
        out_type=jax.ShapeDtypeStruct((seq, D, batch), jnp.float32),
        compiler_params=pltpu.CompilerParams(
            use_tc_tiling_on_sc=False, needs_layout_passes=False),
        scratch_types=scratch,
    )
    def k(x_hbm, tab_hbm, pos_hbm, out_hbm, *sc):
        idx = sc[:_NB]
        rows = sc[_NB:2 * _NB]
        blk = sc[2 * _NB:3 * _NB]
        pos_v = sc[3 * _NB]
        sems = sc[3 * _NB + 1:]
        isem = sems[:_NB]
        gsem = sems[_NB:2 * _NB]
        osem = sems[2 * _NB:]

        wid = lax.axis_index("s") * _NC + lax.axis_index("c")
        pltpu.sync_copy(pos_hbm, pos_v)

        iota = lax.iota(jnp.int32, LANES)
        sidx0 = iota * _PITCH
        off = LANES * _PITCH

        def unit_lb(u):
            g = u * _NW + wid
            return g // nblk, (g % nblk) * _BB

        def fire_idx(u, b):
            l, b0 = unit_lb(u)
            t0 = pl.multiple_of(l * batch + b0, _BB)
            pltpu.async_copy(x_hbm.at[pl.ds(t0, _BB)], idx[b], isem[b])

        def wait_idx(b):
            pltpu.make_async_copy(
                x_hbm.at[pl.ds(0, _BB)], idx[b], isem[b]).wait()

        def fire_gathers(b):
            for j in range(_BB // _GS):
                pltpu.async_copy(
                    tab_hbm.at[idx[b].at[pl.ds(j * _GS, _GS)]],
                    rows[b].at[pl.ds(j * _GS, _GS)],
                    gsem[b],
                )

        def wait_gathers(b):
            pltpu.make_async_copy(
                tab_hbm.at[pl.ds(0, _BB)], rows[b], gsem[b]).wait()

        def transform(u, b):
            rb, ob = rows[b], blk[b]
            l, _ = unit_lb(u)
            pv0 = pos_v[l, pl.ds(0, LANES)]
            pv1 = pos_v[l, pl.ds(LANES, LANES)]

            def body(g, sidx):
                j0 = g * _UNROLL
                for t in range(_UNROLL):
                    j = j0 + t
                    v0 = rb[j, pl.ds(0, LANES)] + pv0
                    v1 = rb[j, pl.ds(LANES, LANES)] + pv1
                    plsc.store_scatter(ob, [sidx], v0)
                    plsc.store_scatter(ob, [sidx + off], v1)
                    sidx = sidx + 1
                return sidx

            lax.fori_loop(0, _BB // _UNROLL, body, sidx0)

        def fire_out(u, b):
            l, b0 = unit_lb(u)

            def fires(d, c):
                pltpu.async_copy(
                    blk[b].at[pl.ds(d * _PITCH, _BB)],
                    out_hbm.at[l, d, pl.ds(b0, _BB)],
                    osem[b],
                )
                return c

            lax.fori_loop(0, D, fires, 0)

        def wait_out(b):
            pltpu.make_async_copy(
                tab_hbm.at[pl.ds(0, _BB)], rows[b], osem[b]).wait()

        def step(u, b, prefetch_idx, first, last):
            wait_gathers(b)
            if prefetch_idx:
                fire_idx(u + 2, b)
            if not last:
                wait_idx(1 - b)
                fire_gathers(1 - b)
            if not first:
                wait_out(b)
            transform(u, b)
            fire_out(u, b)

        fire_idx(0, 0)
        fire_idx(1, 1)
        wait_idx(0)
        fire_gathers(0)

        def main(it, c):
            u0 = it * _NB
            step(u0, 0, True, False, False)
            step(u0 + 1, 1, True, False, False)
            return c

        step(0, 0, True, True, False)
        step(1, 1, True, True, False)
        lax.fori_loop(1, (upw - 2) // _NB, main, 0)
        step(upw - 2, 0, False, False, False)
        step(upw - 1, 1, False, False, True)
        wait_out(0)
        wait_out(1)

    return k(x_lin, table, pos)


def kernel(x, embedding_table, possitional_emb):
    b, l = x.shape
    x_lin = x.T.reshape(b * l)
    out_t = _embed(b, l, x_lin, embedding_table, possitional_emb)
    return out_t.transpose(2, 0, 1)
